# baseline (speedup 1.0000x reference)
import jax
import jax.numpy as jnp
from jax import lax
from jax.experimental import pallas as pl
from jax.experimental.pallas import tpu as pltpu
from jax.experimental.pallas import tpu_sc as plsc

_G = 512
_D = 128
_CH = 256
_TCBLK = 8192
_NW = 32
_GPW = _G // _NW
_NV = _D // 16


def _scores_body(x_ref, w1_ref, b1_ref, w2_ref, o_ref):
    h = jnp.tanh(
        jnp.dot(x_ref[...], w1_ref[...], preferred_element_type=jnp.float32)
        + b1_ref[...])
    sc = lax.dot_general(w2_ref[...], h, (((1,), (1,)), ((), ())),
                         preferred_element_type=jnp.float32)
    o_ref[...] = sc.reshape(_TCBLK // _D, _D)


def _compute_scores(x, W1, b1, W2):
    n = x.shape[0]
    nblk = pl.cdiv(n, _TCBLK)
    rows = _TCBLK // _D
    return pl.pallas_call(
        _scores_body,
        grid=(nblk,),
        in_specs=[
            pl.BlockSpec((_TCBLK, _D), lambda i: (i, 0)),
            pl.BlockSpec((_D, _D), lambda i: (0, 0)),
            pl.BlockSpec((1, _D), lambda i: (0, 0)),
            pl.BlockSpec((1, _D), lambda i: (0, 0)),
        ],
        out_specs=pl.BlockSpec((rows, _D), lambda i: (i, 0)),
        out_shape=jax.ShapeDtypeStruct((nblk * rows, _D), jnp.float32),
    )(x, W1, b1.reshape(1, _D), W2.reshape(1, _D))


def _sc_body(x_hbm, sc_hbm, off_hbm, wgb_hbm, out_hbm,
             xb, sb, offv, wbuf, stg_out, semx, sems):
    n_rows = x_hbm.shape[0] // _D
    wid = lax.axis_index("c") * 16 + lax.axis_index("s")
    gbase = wid * _GPW
    pltpu.sync_copy(off_hbm, offv.at[pl.ds(0, 520)])
    pltpu.sync_copy(wgb_hbm, wbuf.at[pl.ds(0, 264)])

    pv = wbuf[pl.ds(0, 16)]
    a0 = jnp.full((16,), pv[0], dtype=jnp.float32)
    a1 = jnp.full((16,), pv[1], dtype=jnp.float32)
    a2 = jnp.full((16,), pv[2], dtype=jnp.float32)
    mv = jnp.maximum(jnp.maximum(a0, a1), a2)
    e0v = jnp.exp(a0 - mv)
    e1v = jnp.exp(a1 - mv)
    e2v = jnp.exp(a2 - mv)
    esv = e0v + e1v + e2v
    w0v = e0v / esv
    w1v = e1v / esv
    w2v = e2v / esv
    gvs = [wbuf[pl.ds(8 + 16 * j, 16)] for j in range(_NV)]
    bvs = [wbuf[pl.ds(136 + 16 * j, 16)] for j in range(_NV)]

    def _lanesum(vec):
        wbuf[pl.ds(256, 16)] = vec
        wbuf[pl.ds(272, 16)] = jnp.zeros((16,), jnp.float32)
        for sh in (8, 4, 2, 1):
            s = wbuf[pl.ds(256, 16)] + wbuf[pl.ds(256 + sh, 16)]
            wbuf[pl.ds(256, 16)] = s
        red = wbuf[pl.ds(256, 16)]
        return jnp.full((16,), red[0], dtype=jnp.float32)

    t0 = offv[pl.ds(gbase, 16)][0]
    t1 = offv[pl.ds(gbase + _GPW, 16)][0]
    base_a = (t0 // 8) * 8
    ncht = jnp.where(t1 > t0, (t1 - base_a + _CH - 1) // _CH, 0)
    _SB = _CH + 16

    def _chunk_base(k):
        return jnp.minimum(base_a + k * _CH, n_rows - _CH)

    def _issue(k):
        bc = _chunk_base(k)
        par = k % 2
        pltpu.make_async_copy(
            x_hbm.at[pl.ds(bc * _D, _CH * _D)],
            xb.at[pl.ds(par * (_CH * _D), _CH * _D)], semx.at[par]).start()
        pltpu.make_async_copy(
            sc_hbm.at[pl.ds(bc, _CH)],
            sb.at[pl.ds(par * _SB, _CH)], sems.at[par]).start()

    def _wait(k):
        par = k % 2
        pltpu.make_async_copy(
            x_hbm.at[pl.ds(0, _CH * _D)],
            xb.at[pl.ds(par * (_CH * _D), _CH * _D)], semx.at[par]).wait()
        pltpu.make_async_copy(
            sc_hbm.at[pl.ds(0, _CH)],
            sb.at[pl.ds(par * _SB, _CH)], sems.at[par]).wait()

    @pl.when(ncht > 0)
    def _prime():
        _issue(0)

    def _rows(lo, hi, bc, xoff, soff, acc):
        def row_body(r, rc):
            rsum, rmax, rv, rsc = rc
            rl = r - bc
            sv = sb[pl.ds(soff + rl, 16)]
            p = jnp.exp(jnp.full((16,), sv[0], dtype=jnp.float32))
            nsum, nmax, nv = [], [], []
            for j in range(_NV):
                xv = xb[pl.ds(xoff + rl * _D + 16 * j, 16)]
                nsum.append(rsum[j] + xv)
                nmax.append(jnp.maximum(rmax[j], xv))
                nv.append(rv[j] + p * xv)
            return (tuple(nsum), tuple(nmax), tuple(nv), rsc + p)

        return lax.fori_loop(lo, hi, row_body, acc)

    def graph_body(gl, gcarry):
        done, issued = gcarry
        g = gbase + gl
        ov = offv[pl.ds(g, 16)]
        s0 = ov[0]
        s1 = ov[1]
        n = s1 - s0
        c_lo = (s0 - base_a) // _CH
        c_hi_p1 = jnp.where(n > 0, (s1 - 1 - base_a) // _CH + 1, c_lo)

        zeros = jnp.zeros((16,), jnp.float32)
        ninf = jnp.full((16,), -jnp.inf, dtype=jnp.float32)
        acc0 = ((zeros,) * _NV, (ninf,) * _NV, (zeros,) * _NV, zeros,
                done, issued)

        def chunk_body(k, carry):
            asum, amax, av, asc, done, issued = carry
            done = lax.cond(done <= k,
                            lambda: (_wait(k), k + 1)[1],
                            lambda: done)
            issued = lax.cond((issued <= k + 1) & (k + 1 < ncht),
                              lambda: (_issue(k + 1), k + 2)[1],
                              lambda: issued)
            bc = _chunk_base(k)
            lo = jnp.maximum(s0, base_a + k * _CH)
            hi = jnp.minimum(s1, base_a + (k + 1) * _CH)
            par = k % 2
            asum, amax, av, asc = _rows(
                lo, hi, bc, par * (_CH * _D), par * _SB,
                (asum, amax, av, asc))
            return (asum, amax, av, asc, done, issued)

        asum, amax, av, asc, done, issued = lax.fori_loop(
            c_lo, c_hi_p1, chunk_body, acc0)

        nf = jnp.full((16,), n, dtype=jnp.int32).astype(jnp.float32)
        one = jnp.full((16,), 1.0, dtype=jnp.float32)
        invc = one / jnp.maximum(nf, one)
        invs = one / asc
        emb = [w0v * (asum[j] * invc) + w1v * amax[j] + w2v * (av[j] * invs)
               for j in range(_NV)]
        tot = emb[0]
        for j in range(1, _NV):
            tot = tot + emb[j]
        muv = _lanesum(tot) * (1.0 / _D)
        var = jnp.zeros((16,), jnp.float32)
        for j in range(_NV):
            d = emb[j] - muv
            var = var + d * d
        a = _lanesum(var) * (1.0 / _D) + 1e-5
        s = jnp.maximum(a, one)
        for _ in range(18):
            s = 0.5 * (s + a / s)
        y = one / s
        for j in range(_NV):
            stg_out[pl.ds(gl * _D + 16 * j, 16)] = (
                (emb[j] - muv) * y * gvs[j] + bvs[j])
        return (done, issued)

    lax.fori_loop(0, _GPW, graph_body, (0, jnp.where(ncht > 0, 1, 0)))

    pltpu.sync_copy(stg_out, out_hbm.at[pl.ds(gbase * _D, _GPW * _D)])


def _sc_reduce(x, scores1d, off, wgb):
    mesh = plsc.VectorSubcoreMesh(core_axis_name="c", subcore_axis_name="s")
    f = pl.kernel(
        _sc_body,
        out_type=jax.ShapeDtypeStruct((_G * _D,), jnp.float32),
        mesh=mesh,
        scratch_types=[
            pltpu.VMEM((2 * _CH * _D,), jnp.float32),
            pltpu.VMEM((2 * (_CH + 16),), jnp.float32),
            pltpu.VMEM((544,), jnp.int32),
            pltpu.VMEM((288,), jnp.float32),
            pltpu.VMEM((_GPW * _D,), jnp.float32),
            pltpu.SemaphoreType.DMA((2,)),
            pltpu.SemaphoreType.DMA((2,)),
        ],
    )
    return f(x.reshape(-1), scores1d, off, wgb)


def kernel(x, batch, W1, b1, W2, pool_weights, ln_gamma, ln_beta):
    x = x.astype(jnp.float32)
    batch = batch.astype(jnp.int32)
    off = jnp.searchsorted(
        batch, jnp.arange(520, dtype=jnp.int32),
        method="compare_all").astype(jnp.int32)

    scores = _compute_scores(x, W1, b1, W2).reshape(-1)
    wgb = jnp.concatenate([
        pool_weights.astype(jnp.float32),
        jnp.zeros((5,), jnp.float32),
        ln_gamma.astype(jnp.float32),
        ln_beta.astype(jnp.float32),
    ])
    out = _sc_reduce(x, scores, off, wgb)
    return out.reshape(_G, _D)


# speedup vs baseline: 1.0328x; 1.0328x over previous
import jax
import jax.numpy as jnp
from jax import lax
from jax.experimental import pallas as pl
from jax.experimental.pallas import tpu as pltpu
from jax.experimental.pallas import tpu_sc as plsc

_G = 512
_D = 128
_CH = 256
_TCBLK = 8192
_NW = 32
_GPW = _G // _NW
_NV = _D // 16


def _scores_body(x_ref, w1_ref, b1_ref, w2_ref, o_ref):
    h = jnp.tanh(
        jnp.dot(x_ref[...], w1_ref[...], preferred_element_type=jnp.float32)
        + b1_ref[...])
    sc = lax.dot_general(w2_ref[...], h, (((1,), (1,)), ((), ())),
                         preferred_element_type=jnp.float32)
    o_ref[...] = sc.reshape(_TCBLK // _D, _D)


def _compute_scores(x, W1, b1, W2):
    n = x.shape[0]
    nblk = pl.cdiv(n, _TCBLK)
    rows = _TCBLK // _D
    return pl.pallas_call(
        _scores_body,
        grid=(nblk,),
        in_specs=[
            pl.BlockSpec((_TCBLK, _D), lambda i: (i, 0)),
            pl.BlockSpec((_D, _D), lambda i: (0, 0)),
            pl.BlockSpec((1, _D), lambda i: (0, 0)),
            pl.BlockSpec((1, _D), lambda i: (0, 0)),
        ],
        out_specs=pl.BlockSpec((rows, _D), lambda i: (i, 0)),
        out_shape=jax.ShapeDtypeStruct((nblk * rows, _D), jnp.float32),
    )(x, W1, b1.reshape(1, _D), W2.reshape(1, _D))


def _sc_body(x_hbm, sc_hbm, off_hbm, sum_hbm, max_hbm, v_hbm, aux_hbm,
             xb, sb, offv, stg_sum, stg_max, stg_v, stg_aux, semx, sems):
    n_rows = x_hbm.shape[0] // _D
    wid = lax.axis_index("c") * 16 + lax.axis_index("s")
    gbase = wid * _GPW
    pltpu.sync_copy(off_hbm, offv.at[pl.ds(0, 520)])

    t0 = offv[pl.ds(gbase, 16)][0]
    t1 = offv[pl.ds(gbase + _GPW, 16)][0]
    base_a = (t0 // 8) * 8
    ncht = jnp.where(t1 > t0, (t1 - base_a + _CH - 1) // _CH, 0)
    _SB = _CH + 16

    def _chunk_base(k):
        return jnp.minimum(base_a + k * _CH, n_rows - _CH)

    def _issue(k):
        bc = _chunk_base(k)
        par = k % 2
        pltpu.make_async_copy(
            x_hbm.at[pl.ds(bc * _D, _CH * _D)],
            xb.at[pl.ds(par * (_CH * _D), _CH * _D)], semx.at[par]).start()
        pltpu.make_async_copy(
            sc_hbm.at[pl.ds(bc, _CH)],
            sb.at[pl.ds(par * _SB, _CH)], sems.at[par]).start()

    def _wait(k):
        par = k % 2
        pltpu.make_async_copy(
            x_hbm.at[pl.ds(0, _CH * _D)],
            xb.at[pl.ds(par * (_CH * _D), _CH * _D)], semx.at[par]).wait()
        pltpu.make_async_copy(
            sc_hbm.at[pl.ds(0, _CH)],
            sb.at[pl.ds(par * _SB, _CH)], sems.at[par]).wait()

    @pl.when(ncht > 0)
    def _prime():
        _issue(0)

    def _rows(lo, hi, bc, xoff, soff, acc):
        def row_body(r, rc):
            rsum, rmax, rv, rsc = rc
            rl = r - bc
            sv = sb[pl.ds(soff + rl, 16)]
            p = jnp.exp(jnp.full((16,), sv[0], dtype=jnp.float32))
            nsum, nmax, nv = [], [], []
            for j in range(_NV):
                xv = xb[pl.ds(xoff + rl * _D + 16 * j, 16)]
                nsum.append(rsum[j] + xv)
                nmax.append(jnp.maximum(rmax[j], xv))
                nv.append(rv[j] + p * xv)
            return (tuple(nsum), tuple(nmax), tuple(nv), rsc + p)

        return lax.fori_loop(lo, hi, row_body, acc)

    def graph_body(gl, gcarry):
        done, issued = gcarry
        g = gbase + gl
        ov = offv[pl.ds(g, 16)]
        s0 = ov[0]
        s1 = ov[1]
        n = s1 - s0
        c_lo = (s0 - base_a) // _CH
        c_hi_p1 = jnp.where(n > 0, (s1 - 1 - base_a) // _CH + 1, c_lo)

        zeros = jnp.zeros((16,), jnp.float32)
        ninf = jnp.full((16,), -jnp.inf, dtype=jnp.float32)
        acc0 = ((zeros,) * _NV, (ninf,) * _NV, (zeros,) * _NV, zeros,
                done, issued)

        def chunk_body(k, carry):
            asum, amax, av, asc, done, issued = carry
            done = lax.cond(done <= k,
                            lambda: (_wait(k), k + 1)[1],
                            lambda: done)
            issued = lax.cond((issued <= k + 1) & (k + 1 < ncht),
                              lambda: (_issue(k + 1), k + 2)[1],
                              lambda: issued)
            bc = _chunk_base(k)
            lo = jnp.maximum(s0, base_a + k * _CH)
            hi = jnp.minimum(s1, base_a + (k + 1) * _CH)
            par = k % 2
            asum, amax, av, asc = _rows(
                lo, hi, bc, par * (_CH * _D), par * _SB,
                (asum, amax, av, asc))
            return (asum, amax, av, asc, done, issued)

        asum, amax, av, asc, done, issued = lax.fori_loop(
            c_lo, c_hi_p1, chunk_body, acc0)

        for j in range(_NV):
            stg_sum[pl.ds(gl * _D + 16 * j, 16)] = asum[j]
            stg_max[pl.ds(gl * _D + 16 * j, 16)] = amax[j]
            stg_v[pl.ds(gl * _D + 16 * j, 16)] = av[j]
        stg_aux[pl.ds(gl * _D, 16)] = asc
        cntv = jnp.full((16,), n, dtype=jnp.int32).astype(jnp.float32)
        stg_aux[pl.ds(gl * _D + 16, 16)] = cntv
        return (done, issued)

    lax.fori_loop(0, _GPW, graph_body, (0, jnp.where(ncht > 0, 1, 0)))

    pltpu.sync_copy(stg_sum, sum_hbm.at[pl.ds(gbase * _D, _GPW * _D)])
    pltpu.sync_copy(stg_max, max_hbm.at[pl.ds(gbase * _D, _GPW * _D)])
    pltpu.sync_copy(stg_v, v_hbm.at[pl.ds(gbase * _D, _GPW * _D)])
    pltpu.sync_copy(stg_aux, aux_hbm.at[pl.ds(gbase * _D, _GPW * _D)])


def _sc_reduce(x, scores1d, off):
    mesh = plsc.VectorSubcoreMesh(core_axis_name="c", subcore_axis_name="s")
    f = pl.kernel(
        _sc_body,
        out_type=(
            jax.ShapeDtypeStruct((_G * _D,), jnp.float32),
            jax.ShapeDtypeStruct((_G * _D,), jnp.float32),
            jax.ShapeDtypeStruct((_G * _D,), jnp.float32),
            jax.ShapeDtypeStruct((_G * _D,), jnp.float32),
        ),
        mesh=mesh,
        scratch_types=[
            pltpu.VMEM((2 * _CH * _D,), jnp.float32),
            pltpu.VMEM((2 * (_CH + 16),), jnp.float32),
            pltpu.VMEM((544,), jnp.int32),
            pltpu.VMEM((_GPW * _D,), jnp.float32),
            pltpu.VMEM((_GPW * _D,), jnp.float32),
            pltpu.VMEM((_GPW * _D,), jnp.float32),
            pltpu.VMEM((_GPW * _D,), jnp.float32),
            pltpu.SemaphoreType.DMA((2,)),
            pltpu.SemaphoreType.DMA((2,)),
        ],
    )
    return f(x.reshape(-1), scores1d, off)


def _epilogue_body(sum_ref, max_ref, v_ref, aux_ref, w_ref, gam_ref, bet_ref,
                   o_ref):
    s = aux_ref[:, 0:1]
    cnt = aux_ref[:, 16:17]
    mean = sum_ref[...] / jnp.maximum(cnt, 1.0)
    att = v_ref[...] / s
    p0 = w_ref[0, 0]
    p1 = w_ref[0, 1]
    p2 = w_ref[0, 2]
    m = jnp.maximum(jnp.maximum(p0, p1), p2)
    e0 = jnp.exp(p0 - m)
    e1 = jnp.exp(p1 - m)
    e2 = jnp.exp(p2 - m)
    si = 1.0 / (e0 + e1 + e2)
    emb = (e0 * si) * mean + (e1 * si) * max_ref[...] + (e2 * si) * att
    mu = jnp.mean(emb, axis=1, keepdims=True)
    var = jnp.mean((emb - mu) ** 2, axis=1, keepdims=True)
    o_ref[...] = ((emb - mu) * lax.rsqrt(var + 1e-5) * gam_ref[...]
                  + bet_ref[...])


def _epilogue(sum_p, max_p, v_p, aux, wv, gamma, beta):
    return pl.pallas_call(
        _epilogue_body,
        in_specs=[
            pl.BlockSpec((_G, _D), lambda: (0, 0)),
            pl.BlockSpec((_G, _D), lambda: (0, 0)),
            pl.BlockSpec((_G, _D), lambda: (0, 0)),
            pl.BlockSpec((_G, _D), lambda: (0, 0)),
            pl.BlockSpec(memory_space=pltpu.SMEM),
            pl.BlockSpec((1, _D), lambda: (0, 0)),
            pl.BlockSpec((1, _D), lambda: (0, 0)),
        ],
        out_specs=pl.BlockSpec((_G, _D), lambda: (0, 0)),
        out_shape=jax.ShapeDtypeStruct((_G, _D), jnp.float32),
    )(sum_p, max_p, v_p, aux, wv, gamma, beta)


def kernel(x, batch, W1, b1, W2, pool_weights, ln_gamma, ln_beta):
    x = x.astype(jnp.float32)
    batch = batch.astype(jnp.int32)
    off = jnp.searchsorted(
        batch, jnp.arange(520, dtype=jnp.int32),
        method="compare_all").astype(jnp.int32)

    scores = _compute_scores(x, W1, b1, W2).reshape(-1)
    sum_p, max_p, v_p, aux = _sc_reduce(x, scores, off)
    sum_p = sum_p.reshape(_G, _D)
    max_p = max_p.reshape(_G, _D)
    v_p = v_p.reshape(_G, _D)
    aux = aux.reshape(_G, _D)

    wv = pool_weights.astype(jnp.float32).reshape(1, 3)
    return _epilogue(sum_p, max_p, v_p, aux,
                     wv, ln_gamma.reshape(1, _D), ln_beta.reshape(1, _D))
